# baseline (speedup 1.0000x reference)
import functools

import jax
import jax.numpy as jnp
from jax import lax
from jax.experimental import pallas as pl
from jax.experimental.pallas import tpu as pltpu
from jax.experimental.pallas import tpu_sc as plsc


def _gather_body(n_rows, n_chunks, chunk, n_cores, idx_hbm, table_hbm,
                 out_hbm, idx_v, rows_v, sem):
    wid = lax.axis_index("s") * n_cores + lax.axis_index("c")
    base = wid * (n_chunks * chunk)

    def body(c, carry):
        off = base + c * chunk
        pltpu.sync_copy(idx_hbm.at[pl.ds(off, chunk)], idx_v)
        pltpu.async_copy(table_hbm.at[idx_v], rows_v, sem).wait()
        pltpu.sync_copy(rows_v, out_hbm.at[pl.ds(off, chunk)])
        return carry

    lax.fori_loop(0, n_chunks, body, 0)


def _sc_gather(table, idx):
    n, = idx.shape
    _, e = table.shape
    info = plsc.get_sparse_core_info()
    nc, ns = info.num_cores, info.num_subcores
    nw = nc * ns
    chunk = 128
    assert n % (nw * chunk) == 0, (n, nw, chunk)
    n_chunks = n // (nw * chunk)
    mesh = plsc.VectorSubcoreMesh(core_axis_name="c", subcore_axis_name="s")
    kern = functools.partial(
        pl.kernel,
        mesh=mesh,
        out_type=jax.ShapeDtypeStruct((n, e), jnp.float32),
        scratch_types=[
            pltpu.VMEM((chunk,), jnp.int32),
            pltpu.VMEM((chunk, e), jnp.float32),
            pltpu.SemaphoreType.DMA,
        ],
        compiler_params=pltpu.CompilerParams(use_tc_tiling_on_sc=False),
    )(functools.partial(_gather_body, n, n_chunks, chunk, nc))
    return kern(idx, table)


def _rnn_body(t_steps, xe_ref, wih_ref, whh_ref, b_ref, out_ref, h_ref):
    t = pl.program_id(0)

    @pl.when(t == 0)
    def _init():
        h_ref[...] = jnp.zeros_like(h_ref)

    x = xe_ref[0]
    h = h_ref[...]
    z = (jnp.dot(x, wih_ref[...], preferred_element_type=jnp.float32)
         + jnp.dot(h, whh_ref[...], preferred_element_type=jnp.float32)
         + b_ref[...])
    hn = jnp.tanh(z)
    h_ref[...] = hn

    @pl.when(t == t_steps - 1)
    def _out():
        out_ref[...] = hn


def _tc_rnn(xe, wih_t, whh_t, bias):
    t_steps, b, e = xe.shape
    h = whh_t.shape[0]
    return pl.pallas_call(
        functools.partial(_rnn_body, t_steps),
        grid=(t_steps,),
        in_specs=[
            pl.BlockSpec((1, b, e), lambda t: (t, 0, 0)),
            pl.BlockSpec((e, h), lambda t: (0, 0)),
            pl.BlockSpec((h, h), lambda t: (0, 0)),
            pl.BlockSpec((1, h), lambda t: (0, 0)),
        ],
        out_specs=pl.BlockSpec((b, h), lambda t: (0, 0)),
        out_shape=jax.ShapeDtypeStruct((b, h), jnp.float32),
        scratch_shapes=[pltpu.VMEM((b, h), jnp.float32)],
        compiler_params=pltpu.CompilerParams(
            dimension_semantics=("arbitrary",)),
    )(xe, wih_t, whh_t, bias)


def kernel(source, emb_table, W_ih, W_hh, b_ih, b_hh):
    b, t = source.shape
    v, e = emb_table.shape
    h = W_hh.shape[0]
    idx = source.T.reshape(-1).astype(jnp.int32)
    embedded = _sc_gather(emb_table, idx)
    xe = embedded.reshape(t, b, e)
    bias = (b_ih + b_hh).reshape(1, h)
    hidden = _tc_rnn(xe, W_ih.T, W_hh.T, bias)
    return hidden[None, :, :]


# speedup vs baseline: 1.1628x; 1.1628x over previous
import functools

import jax
import jax.numpy as jnp
from jax import lax
from jax.experimental import pallas as pl
from jax.experimental.pallas import tpu as pltpu
from jax.experimental.pallas import tpu_sc as plsc

_CHUNK = 128
_PADW = 128


def _gather_body(n_chunks, n_cores, idx_hbm, table_hbm, out_hbm,
                 idx_v, buf0, buf1, sem0, sem1):
    wid = lax.axis_index("s") * n_cores + lax.axis_index("c")
    base = wid * (n_chunks * _CHUNK)
    pltpu.sync_copy(idx_hbm.at[wid], idx_v)

    def fire(c, buf, sem):
        pltpu.async_copy(table_hbm.at[idx_v.at[c]], buf, sem)

    def drain(c, buf, sem):
        pltpu.make_async_copy(table_hbm.at[idx_v.at[c]], buf, sem).wait()

    def store(c, buf):
        pltpu.sync_copy(
            buf, out_hbm.at[pl.ds(base + c * _CHUNK, _CHUNK), pl.ds(0, 64)])

    fire(0, buf0, sem0)

    def body(i, carry):
        c0 = 2 * i
        fire(c0 + 1, buf1, sem1)
        drain(c0, buf0, sem0)
        store(c0, buf0)

        @pl.when(i + 1 < n_chunks // 2)
        def _():
            fire(c0 + 2, buf0, sem0)

        drain(c0 + 1, buf1, sem1)
        store(c0 + 1, buf1)
        return carry

    lax.fori_loop(0, n_chunks // 2, body, 0)


def _sc_gather(table, idx):
    n, = idx.shape
    info = plsc.get_sparse_core_info()
    nc, ns = info.num_cores, info.num_subcores
    nw = nc * ns
    assert n % (nw * _CHUNK) == 0, (n, nw, _CHUNK)
    n_chunks = n // (nw * _CHUNK)
    assert n_chunks % 2 == 0
    idx3 = idx.reshape(nw, n_chunks, _CHUNK)
    mesh = plsc.VectorSubcoreMesh(core_axis_name="c", subcore_axis_name="s")
    kern = functools.partial(
        pl.kernel,
        mesh=mesh,
        out_type=jax.ShapeDtypeStruct((n, _PADW), jnp.float32),
        scratch_types=[
            pltpu.VMEM((n_chunks, _CHUNK), jnp.int32),
            pltpu.VMEM((_CHUNK, 64), jnp.float32),
            pltpu.VMEM((_CHUNK, 64), jnp.float32),
            pltpu.SemaphoreType.DMA,
            pltpu.SemaphoreType.DMA,
        ],
        compiler_params=pltpu.CompilerParams(use_tc_tiling_on_sc=False),
    )(functools.partial(_gather_body, n_chunks, nc))
    return kern(idx3, table)


def _rnn_body(t_steps, e, xe_ref, wih_ref, whh_ref, b_ref, out_ref, h_ref):
    t = pl.program_id(0)

    @pl.when(t == 0)
    def _init():
        h_ref[...] = jnp.zeros_like(h_ref)

    x = xe_ref[0][:, :e]
    h = h_ref[...]
    z = (jnp.dot(x, wih_ref[...], preferred_element_type=jnp.float32)
         + jnp.dot(h, whh_ref[...], preferred_element_type=jnp.float32)
         + b_ref[...])
    hn = jnp.tanh(z)
    h_ref[...] = hn

    @pl.when(t == t_steps - 1)
    def _out():
        out_ref[...] = hn


def _tc_rnn(xe, wih_t, whh_t, bias):
    t_steps, b, _ = xe.shape
    e, h = wih_t.shape
    return pl.pallas_call(
        functools.partial(_rnn_body, t_steps, e),
        grid=(t_steps,),
        in_specs=[
            pl.BlockSpec((1, b, _PADW), lambda t: (t, 0, 0)),
            pl.BlockSpec((e, h), lambda t: (0, 0)),
            pl.BlockSpec((h, h), lambda t: (0, 0)),
            pl.BlockSpec((1, h), lambda t: (0, 0)),
        ],
        out_specs=pl.BlockSpec((b, h), lambda t: (0, 0)),
        out_shape=jax.ShapeDtypeStruct((b, h), jnp.float32),
        scratch_shapes=[pltpu.VMEM((b, h), jnp.float32)],
        compiler_params=pltpu.CompilerParams(
            dimension_semantics=("arbitrary",)),
    )(xe, wih_t, whh_t, bias)


def kernel(source, emb_table, W_ih, W_hh, b_ih, b_hh):
    b, t = source.shape
    v, e = emb_table.shape
    h = W_hh.shape[0]
    idx = source.T.reshape(-1).astype(jnp.int32)
    embedded = _sc_gather(emb_table, idx)
    xe = embedded.reshape(t, b, _PADW)
    bias = (b_ih + b_hh).reshape(1, h)
    hidden = _tc_rnn(xe, W_ih.T, W_hh.T, bias)
    return hidden[None, :, :]


# speedup vs baseline: 1.7558x; 1.5100x over previous
import functools

import jax
import jax.numpy as jnp
from jax import lax
from jax.experimental import pallas as pl
from jax.experimental.pallas import tpu as pltpu
from jax.experimental.pallas import tpu_sc as plsc

_CHUNK = 128
_PADW = 128


def _gather_body(n_chunks, n_cores, idx_hbm, table_hbm, out_hbm,
                 idx_v, buf0, buf1, sem0, sem1):
    wid = lax.axis_index("s") * n_cores + lax.axis_index("c")
    base = wid * (n_chunks * _CHUNK)
    pltpu.sync_copy(idx_hbm.at[wid], idx_v)

    def fire(c, buf, sem):
        pltpu.async_copy(table_hbm.at[idx_v.at[c]], buf, sem)

    def drain(c, buf, sem):
        pltpu.make_async_copy(table_hbm.at[idx_v.at[c]], buf, sem).wait()

    def store(c, buf):
        pltpu.sync_copy(
            buf, out_hbm.at[pl.ds(base + c * _CHUNK, _CHUNK), pl.ds(0, 64)])

    fire(0, buf0, sem0)

    def body(i, carry):
        c0 = 2 * i
        fire(c0 + 1, buf1, sem1)
        drain(c0, buf0, sem0)
        store(c0, buf0)

        @pl.when(i + 1 < n_chunks // 2)
        def _():
            fire(c0 + 2, buf0, sem0)

        drain(c0 + 1, buf1, sem1)
        store(c0 + 1, buf1)
        return carry

    lax.fori_loop(0, n_chunks // 2, body, 0)


def _sc_gather(table, idx):
    n, = idx.shape
    info = plsc.get_sparse_core_info()
    nc, ns = info.num_cores, info.num_subcores
    nw = nc * ns
    assert n % (nw * _CHUNK) == 0, (n, nw, _CHUNK)
    n_chunks = n // (nw * _CHUNK)
    assert n_chunks % 2 == 0
    idx3 = idx.reshape(nw, n_chunks, _CHUNK)
    mesh = plsc.VectorSubcoreMesh(core_axis_name="c", subcore_axis_name="s")
    kern = functools.partial(
        pl.kernel,
        mesh=mesh,
        out_type=jax.ShapeDtypeStruct((n, _PADW), jnp.float32),
        scratch_types=[
            pltpu.VMEM((n_chunks, _CHUNK), jnp.int32),
            pltpu.VMEM((_CHUNK, 64), jnp.float32),
            pltpu.VMEM((_CHUNK, 64), jnp.float32),
            pltpu.SemaphoreType.DMA,
            pltpu.SemaphoreType.DMA,
        ],
        compiler_params=pltpu.CompilerParams(use_tc_tiling_on_sc=False),
    )(functools.partial(_gather_body, n_chunks, nc))
    return kern(idx3, table)


_TPW = 4096


def _tp_body(in_ref, out_ref):
    x = in_ref[...]
    h = _TPW // 2
    out_ref[...] = jnp.concatenate([x[:, :h].T, x[:, h:].T], axis=1)


def _tc_transpose(tbl_t):
    e, v = tbl_t.shape
    grid = (v + _TPW - 1) // _TPW
    n_out = grid * (_TPW // 2)
    return pl.pallas_call(
        _tp_body,
        grid=(grid,),
        in_specs=[pl.BlockSpec((e, _TPW), lambda i: (0, i))],
        out_specs=pl.BlockSpec((_TPW // 2, 2 * e), lambda i: (i, 0)),
        out_shape=jax.ShapeDtypeStruct((n_out, 2 * e), jnp.float32),
        compiler_params=pltpu.CompilerParams(
            dimension_semantics=("arbitrary",)),
    )(tbl_t)


def _rnn_body(t_steps, e, xe_ref, wih_ref, whh_ref, b_ref, out_ref, h_ref):
    t = pl.program_id(0)

    @pl.when(t == 0)
    def _init():
        h_ref[...] = jnp.zeros_like(h_ref)

    x = xe_ref[0][:, :e]
    h = h_ref[...]
    z = (jnp.dot(x, wih_ref[...], preferred_element_type=jnp.float32)
         + jnp.dot(h, whh_ref[...], preferred_element_type=jnp.float32)
         + b_ref[...])
    hn = jnp.tanh(z)
    h_ref[...] = hn

    @pl.when(t == t_steps - 1)
    def _out():
        out_ref[...] = hn


def _tc_rnn(xe, wih_t, whh_t, bias):
    t_steps, b, _ = xe.shape
    e, h = wih_t.shape
    return pl.pallas_call(
        functools.partial(_rnn_body, t_steps, e),
        grid=(t_steps,),
        in_specs=[
            pl.BlockSpec((1, b, _PADW), lambda t: (t, 0, 0)),
            pl.BlockSpec((e, h), lambda t: (0, 0)),
            pl.BlockSpec((h, h), lambda t: (0, 0)),
            pl.BlockSpec((1, h), lambda t: (0, 0)),
        ],
        out_specs=pl.BlockSpec((b, h), lambda t: (0, 0)),
        out_shape=jax.ShapeDtypeStruct((b, h), jnp.float32),
        scratch_shapes=[pltpu.VMEM((b, h), jnp.float32)],
        compiler_params=pltpu.CompilerParams(
            dimension_semantics=("arbitrary",)),
    )(xe, wih_t, whh_t, bias)


def kernel(source, emb_table, W_ih, W_hh, b_ih, b_hh):
    b, t = source.shape
    v, e = emb_table.shape
    h = W_hh.shape[0]
    idx = source.T.reshape(-1).astype(jnp.int32)
    packed = _tc_transpose(emb_table.T)
    tbl_lin = packed.reshape(packed.shape[0] * 2, e)
    half = _TPW // 2
    g, r = idx // _TPW, idx % _TPW
    jdx = 2 * (g * half + (r % half)) + (r // half)
    embedded = _sc_gather(tbl_lin, jdx)
    xe = embedded.reshape(t, b, _PADW)
    bias = (b_ih + b_hh).reshape(1, h)
    hidden = _tc_rnn(xe, W_ih.T, W_hh.T, bias)
    return hidden[None, :, :]


# speedup vs baseline: 2.1758x; 1.2392x over previous
import functools

import jax
import jax.numpy as jnp
from jax import lax
from jax.experimental import pallas as pl
from jax.experimental.pallas import tpu as pltpu
from jax.experimental.pallas import tpu_sc as plsc

_CHUNK = 128
_PADW = 128


def _gather_body(n_chunks, n_cores, idx_hbm, table_hbm, out_hbm,
                 idx_v, buf0, buf1, sem0, sem1):
    wid = lax.axis_index("s") * n_cores + lax.axis_index("c")
    base = wid * (n_chunks * _CHUNK)
    pltpu.sync_copy(idx_hbm.at[wid], idx_v)

    def fire(c, buf, sem):
        pltpu.async_copy(table_hbm.at[idx_v.at[c]], buf, sem)

    def drain(c, buf, sem):
        pltpu.make_async_copy(table_hbm.at[idx_v.at[c]], buf, sem).wait()

    def store(c, buf):
        pltpu.sync_copy(
            buf, out_hbm.at[pl.ds(base + c * _CHUNK, _CHUNK), pl.ds(0, 64)])

    fire(0, buf0, sem0)

    def body(i, carry):
        c0 = 2 * i
        fire(c0 + 1, buf1, sem1)
        drain(c0, buf0, sem0)
        store(c0, buf0)

        @pl.when(i + 1 < n_chunks // 2)
        def _():
            fire(c0 + 2, buf0, sem0)

        drain(c0 + 1, buf1, sem1)
        store(c0 + 1, buf1)
        return carry

    lax.fori_loop(0, n_chunks // 2, body, 0)


def _sc_gather(table, idx):
    n, = idx.shape
    info = plsc.get_sparse_core_info()
    nc, ns = info.num_cores, info.num_subcores
    nw = nc * ns
    assert n % (nw * _CHUNK) == 0, (n, nw, _CHUNK)
    n_chunks = n // (nw * _CHUNK)
    assert n_chunks % 2 == 0
    idx3 = idx.reshape(nw, n_chunks, _CHUNK)
    mesh = plsc.VectorSubcoreMesh(core_axis_name="c", subcore_axis_name="s")
    kern = functools.partial(
        pl.kernel,
        mesh=mesh,
        out_type=jax.ShapeDtypeStruct((n, _PADW), jnp.float32),
        scratch_types=[
            pltpu.VMEM((n_chunks, _CHUNK), jnp.int32),
            pltpu.VMEM((_CHUNK, 64), jnp.float32),
            pltpu.VMEM((_CHUNK, 64), jnp.float32),
            pltpu.SemaphoreType.DMA,
            pltpu.SemaphoreType.DMA,
        ],
        compiler_params=pltpu.CompilerParams(use_tc_tiling_on_sc=False),
    )(functools.partial(_gather_body, n_chunks, nc))
    return kern(idx3, table)


_TPW = 4096


def _tp_body(in_ref, out_ref):
    x = in_ref[...]
    h = _TPW // 2
    stacked = jnp.concatenate([x[:, :h], x[:, h:]], axis=0)
    out_ref[...] = stacked.T


def _tc_transpose(tbl_t):
    e, v = tbl_t.shape
    half = _TPW // 2
    grid = (v + _TPW - 1) // _TPW
    n_out = grid * half
    return pl.pallas_call(
        _tp_body,
        grid=(grid,),
        in_specs=[pl.BlockSpec((e, _TPW), lambda i: (0, i))],
        out_specs=pl.BlockSpec((half, 2 * e), lambda i: (i, 0)),
        out_shape=jax.ShapeDtypeStruct((n_out, 2 * e), jnp.float32),
        compiler_params=pltpu.CompilerParams(
            dimension_semantics=("arbitrary",)),
    )(tbl_t)


_TSTEP = 2


def _rnn_body(n_grid, e, xe_ref, wih_ref, whh_ref, b_ref, out_ref, h_ref):
    t = pl.program_id(0)

    @pl.when(t == 0)
    def _init():
        h_ref[...] = jnp.zeros_like(h_ref)

    h = h_ref[...]
    wih = wih_ref[...]
    whh = whh_ref[...]
    b = b_ref[...]
    for s in range(_TSTEP):
        x = xe_ref[s][:, :e]
        z = (jnp.dot(x, wih, preferred_element_type=jnp.float32)
             + jnp.dot(h, whh, preferred_element_type=jnp.float32)
             + b)
        h = jnp.tanh(z)
    h_ref[...] = h

    @pl.when(t == n_grid - 1)
    def _out():
        out_ref[...] = h


def _tc_rnn(xe, wih_t, whh_t, bias):
    t_steps, b, _ = xe.shape
    e, h = wih_t.shape
    n_grid = t_steps // _TSTEP
    return pl.pallas_call(
        functools.partial(_rnn_body, n_grid, e),
        grid=(n_grid,),
        in_specs=[
            pl.BlockSpec((_TSTEP, b, _PADW), lambda t: (t, 0, 0)),
            pl.BlockSpec((e, h), lambda t: (0, 0)),
            pl.BlockSpec((h, h), lambda t: (0, 0)),
            pl.BlockSpec((1, h), lambda t: (0, 0)),
        ],
        out_specs=pl.BlockSpec((b, h), lambda t: (0, 0)),
        out_shape=jax.ShapeDtypeStruct((b, h), jnp.float32),
        scratch_shapes=[pltpu.VMEM((b, h), jnp.float32)],
        compiler_params=pltpu.CompilerParams(
            dimension_semantics=("arbitrary",)),
    )(xe, wih_t, whh_t, bias)


def kernel(source, emb_table, W_ih, W_hh, b_ih, b_hh):
    b, t = source.shape
    v, e = emb_table.shape
    h = W_hh.shape[0]
    idx = source.T.reshape(-1).astype(jnp.int32)
    packed = _tc_transpose(emb_table.T)
    tbl_lin = packed.reshape(packed.shape[0] * 2, e)
    half = _TPW // 2
    g, r = idx // _TPW, idx % _TPW
    jdx = 2 * (g * half + (r % half)) + (r // half)
    embedded = _sc_gather(tbl_lin, jdx)
    xe = embedded.reshape(t, b, _PADW)
    bias = (b_ih + b_hh).reshape(1, h)
    hidden = _tc_rnn(xe, W_ih.T, W_hh.T, bias)
    return hidden[None, :, :]


# speedup vs baseline: 2.8320x; 1.3016x over previous
import functools

import jax
import jax.numpy as jnp
from jax import lax
from jax.experimental import pallas as pl
from jax.experimental.pallas import tpu as pltpu
from jax.experimental.pallas import tpu_sc as plsc

_CHUNK = 128
_PADW = 128


def _gather_body(n_chunks, n_cores, idx_hbm, table_hbm, out_hbm,
                 idx_v, buf0, buf1, sem0, sem1):
    wid = lax.axis_index("s") * n_cores + lax.axis_index("c")
    base = wid * (n_chunks * _CHUNK)
    pltpu.sync_copy(idx_hbm.at[wid], idx_v)

    def fire(c, buf, sem):
        pltpu.async_copy(table_hbm.at[idx_v.at[c]], buf, sem)

    def drain(c, buf, sem):
        pltpu.make_async_copy(table_hbm.at[idx_v.at[c]], buf, sem).wait()

    def store(c, buf):
        pltpu.sync_copy(
            buf, out_hbm.at[pl.ds(base + c * _CHUNK, _CHUNK), pl.ds(0, 64)])

    fire(0, buf0, sem0)

    def body(i, carry):
        c0 = 2 * i
        fire(c0 + 1, buf1, sem1)
        drain(c0, buf0, sem0)
        store(c0, buf0)

        @pl.when(i + 1 < n_chunks // 2)
        def _():
            fire(c0 + 2, buf0, sem0)

        drain(c0 + 1, buf1, sem1)
        store(c0 + 1, buf1)
        return carry

    lax.fori_loop(0, n_chunks // 2, body, 0)


def _sc_gather(table, idx):
    n, = idx.shape
    info = plsc.get_sparse_core_info()
    nc, ns = info.num_cores, info.num_subcores
    nw = nc * ns
    assert n % (nw * _CHUNK) == 0, (n, nw, _CHUNK)
    n_chunks = n // (nw * _CHUNK)
    assert n_chunks % 2 == 0
    idx3 = idx.reshape(nw, n_chunks, _CHUNK)
    mesh = plsc.VectorSubcoreMesh(core_axis_name="c", subcore_axis_name="s")
    kern = functools.partial(
        pl.kernel,
        mesh=mesh,
        out_type=jax.ShapeDtypeStruct((n, _PADW), jnp.float32),
        scratch_types=[
            pltpu.VMEM((n_chunks, _CHUNK), jnp.int32),
            pltpu.VMEM((_CHUNK, 64), jnp.float32),
            pltpu.VMEM((_CHUNK, 64), jnp.float32),
            pltpu.SemaphoreType.DMA,
            pltpu.SemaphoreType.DMA,
        ],
        compiler_params=pltpu.CompilerParams(use_tc_tiling_on_sc=False),
    )(functools.partial(_gather_body, n_chunks, nc))
    return kern(idx3, table)


_TPW = 8192


def _tp_body(in_ref, out_ref):
    x = in_ref[...]
    h = _TPW // 2
    stacked = jnp.concatenate([x[:, :h], x[:, h:]], axis=0)
    out_ref[...] = stacked.T


def _tc_transpose(tbl_t):
    e, v = tbl_t.shape
    half = _TPW // 2
    grid = (v + _TPW - 1) // _TPW
    n_out = grid * half
    return pl.pallas_call(
        _tp_body,
        grid=(grid,),
        in_specs=[pl.BlockSpec((e, _TPW), lambda i: (0, i))],
        out_specs=pl.BlockSpec((half, 2 * e), lambda i: (i, 0)),
        out_shape=jax.ShapeDtypeStruct((n_out, 2 * e), jnp.float32),
        compiler_params=pltpu.CompilerParams(
            dimension_semantics=("arbitrary",)),
    )(tbl_t)


_TSTEP = 4


def _rnn_body(n_grid, e, xe_ref, wih_ref, whh_ref, b_ref, out_ref, h_ref):
    t = pl.program_id(0)

    @pl.when(t == 0)
    def _init():
        h_ref[...] = jnp.zeros_like(h_ref)

    h = h_ref[...]
    wih = wih_ref[...]
    whh = whh_ref[...]
    b = b_ref[...]
    for s in range(_TSTEP):
        x = xe_ref[s][:, :e]
        z = (jnp.dot(x, wih, preferred_element_type=jnp.float32)
             + jnp.dot(h, whh, preferred_element_type=jnp.float32)
             + b)
        h = jnp.tanh(z)
    h_ref[...] = h

    @pl.when(t == n_grid - 1)
    def _out():
        out_ref[...] = h


def _tc_rnn(xe, wih_t, whh_t, bias):
    t_steps, b, _ = xe.shape
    e, h = wih_t.shape
    n_grid = t_steps // _TSTEP
    return pl.pallas_call(
        functools.partial(_rnn_body, n_grid, e),
        grid=(n_grid,),
        in_specs=[
            pl.BlockSpec((_TSTEP, b, _PADW), lambda t: (t, 0, 0)),
            pl.BlockSpec((e, h), lambda t: (0, 0)),
            pl.BlockSpec((h, h), lambda t: (0, 0)),
            pl.BlockSpec((1, h), lambda t: (0, 0)),
        ],
        out_specs=pl.BlockSpec((b, h), lambda t: (0, 0)),
        out_shape=jax.ShapeDtypeStruct((b, h), jnp.float32),
        scratch_shapes=[pltpu.VMEM((b, h), jnp.float32)],
        compiler_params=pltpu.CompilerParams(
            dimension_semantics=("arbitrary",)),
    )(xe, wih_t, whh_t, bias)


def kernel(source, emb_table, W_ih, W_hh, b_ih, b_hh):
    b, t = source.shape
    v, e = emb_table.shape
    h = W_hh.shape[0]
    idx = source.T.reshape(-1).astype(jnp.int32)
    packed = _tc_transpose(emb_table.T)
    tbl_lin = packed.reshape(packed.shape[0] * 2, e)
    half = _TPW // 2
    g, r = idx // _TPW, idx % _TPW
    jdx = 2 * (g * half + (r % half)) + (r // half)
    embedded = _sc_gather(tbl_lin, jdx)
    xe = embedded.reshape(t, b, _PADW)
    bias = (b_ih + b_hh).reshape(1, h)
    hidden = _tc_rnn(xe, W_ih.T, W_hh.T, bias)
    return hidden[None, :, :]
